# baseline (speedup 1.0000x reference)
import functools

import jax
import jax.numpy as jnp
from jax import lax
from jax.experimental import pallas as pl
from jax.experimental.pallas import tpu as pltpu
from jax.experimental.pallas import tpu_sc as plsc

N_NODES = 10000
N_EDGES = 160000
D_FEAT = 256

NC, NS, L = 2, 16, 16
NW = NC * NS
EPW = N_EDGES // NW
B = 32
NFULL = EPW // B
TAIL = EPW - NFULL * B
GROUPS = B // L
UNROLL = 8
DW = D_FEAT // 2
RPT = N_NODES // NS
RSTAGE = 25


def _rsqrt_newton(ss):
    ib = lax.bitcast_convert_type(ss, jnp.int32)
    seed = jnp.int32(0x5F3759DF) - lax.shift_right_logical(ib, 1)
    y = lax.bitcast_convert_type(seed, jnp.float32)
    for _ in range(3):
        y = y * (1.5 - 0.5 * ss * y * y)
    return y


def _group_sumsq(rows, row_idx, lane):
    def feat_body(k, carry):
        s0, s1 = carry
        col0 = lane + (k * UNROLL)
        for dd in range(UNROLL):
            col = jnp.bitwise_and(col0 + dd, DW - 1)
            w0 = plsc.load_gather(rows, [row_idx, col])
            w1 = plsc.load_gather(rows, [row_idx + 1, col])
            d = plsc.bitcast(w0, jnp.bfloat16) - plsc.bitcast(w1, jnp.bfloat16)
            d2 = plsc.bitcast(d * d, jnp.int32)
            lo = lax.bitcast_convert_type(jnp.left_shift(d2, 16), jnp.float32)
            hi = lax.bitcast_convert_type(
                jnp.bitwise_and(d2, jnp.int32(-65536)), jnp.float32)
            s0 = s0 + lo
            s1 = s1 + hi
        return s0, s1

    z = jnp.zeros((L,), jnp.float32)
    s0, s1 = lax.fori_loop(0, DW // UNROLL, feat_body, (z, z))
    return s0 + s1


def _edge_sqerr(ss, sd):
    ss = jnp.maximum(ss, 1e-30)
    dist = ss * _rsqrt_newton(ss)
    r = sd - dist
    return r * r


_mesh = plsc.VectorSubcoreMesh(core_axis_name="c", subcore_axis_name="s")


@functools.partial(
    pl.kernel,
    mesh=_mesh,
    compiler_params=pltpu.CompilerParams(use_tc_tiling_on_sc=False,
                                         needs_layout_passes=False),
    out_type=jax.ShapeDtypeStruct((NW, L), jnp.float32),
    scratch_types=[
        pltpu.VMEM_SHARED((N_NODES, DW), jnp.int32),
        pltpu.VMEM((2 * EPW + 32,), jnp.int32),
        pltpu.VMEM((EPW + 16,), jnp.float32),
        pltpu.VMEM((2 * B, DW), jnp.int32),
        pltpu.VMEM((2 * B, DW), jnp.int32),
        pltpu.VMEM((RSTAGE, D_FEAT), jnp.float32),
        pltpu.VMEM((RSTAGE, DW), jnp.int32),
        pltpu.VMEM((L,), jnp.float32),
        pltpu.SemaphoreType.DMA,
        pltpu.SemaphoreType.DMA,
        pltpu.SemaphoreType.DMA,
        pltpu.SemaphoreType.DMA,
    ],
)
def _edge_partials(x_hbm, idx_hbm, sd_hbm, out_hbm,
                   table, idx_all, sd_all,
                   rowsa, rowsb, stage, wstage, accv,
                   s0a, s1a, s0b, s1b):
    cid = lax.axis_index("c")
    sid = lax.axis_index("s")
    wid = sid * NC + cid
    e0 = pl.multiple_of(wid * EPW, 8)
    lane = lax.iota(jnp.int32, L)

    tbase = sid * RPT
    for cc in range(RPT // RSTAGE):
        nb = tbase + cc * RSTAGE
        pltpu.sync_copy(x_hbm.at[pl.ds(nb, RSTAGE)], stage)

        def row_body(r, carry):
            for k in range(D_FEAT // 32):
                a = stage[r, pl.ds(32 * k, 16)]
                b = stage[r, pl.ds(32 * k + 16, 16)]
                wv = plsc.bitcast(
                    plsc.pack(a, b, format=plsc.PackFormat.INTERLEAVED),
                    jnp.int32)
                wstage[r, pl.ds(16 * k, 16)] = wv
            return carry

        lax.fori_loop(0, RSTAGE, row_body, 0)
        pltpu.sync_copy(wstage, table.at[pl.ds(nb, RSTAGE)])
    plsc.subcore_barrier()

    rows = (rowsa, rowsb)
    sems = ((s0a, s1a), (s0b, s1b))

    pltpu.sync_copy(idx_hbm.at[pl.ds(2 * e0, 2 * EPW)],
                    idx_all.at[pl.ds(0, 2 * EPW)])
    pltpu.sync_copy(sd_hbm.at[pl.ds(e0, EPW)], sd_all.at[pl.ds(0, EPW)])

    def issue(t, parity, n):
        r = rows[parity]
        sm0, sm1 = sems[parity]
        pltpu.async_copy(table.at[idx_all.at[pl.ds(t * 2 * B, n)]],
                         r.at[pl.ds(0, n)], sm0)
        pltpu.async_copy(table.at[idx_all.at[pl.ds(t * 2 * B + n, n)]],
                         r.at[pl.ds(n, n)], sm1)

    def wait(t, parity, n):
        r = rows[parity]
        sm0, sm1 = sems[parity]
        pltpu.make_async_copy(table.at[idx_all.at[pl.ds(t * 2 * B, n)]],
                              r.at[pl.ds(0, n)], sm0).wait()
        pltpu.make_async_copy(table.at[idx_all.at[pl.ds(t * 2 * B + n, n)]],
                              r.at[pl.ds(n, n)], sm1).wait()

    def compute(t, parity, acc):
        r = rows[parity]
        for g in range(GROUPS):
            row_idx = (lane + (g * L)) * 2
            ss = _group_sumsq(r, row_idx, lane)
            sd = sd_all[pl.ds(t * B + g * L, L)]
            acc = acc + _edge_sqerr(ss, sd)
        return acc

    issue(0, 0, B)

    def outer_body(i, acc):
        tb = i * 2
        for par in range(2):
            t = tb + par

            @pl.when(t + 1 < NFULL)
            def _():
                issue(t + 1, 1 - par, B)

            wait(t, par, B)
            acc = compute(t, par, acc)
        return acc

    acc = lax.fori_loop(0, NFULL // 2, outer_body,
                        jnp.zeros((L,), jnp.float32))

    if NFULL % 2:
        t_last = NFULL - 1
        wait(t_last, t_last % 2, B)
        acc = compute(t_last, t_last % 2, acc)

    tpar = NFULL % 2
    issue(NFULL, tpar, TAIL)
    wait(NFULL, tpar, TAIL)
    row_idx = jnp.bitwise_and(lane, TAIL - 1) * 2
    ss = _group_sumsq(rows[tpar], row_idx, lane)
    sd = sd_all[pl.ds(NFULL * B, L)]
    sq = _edge_sqerr(ss, sd)
    acc = acc + jnp.where(lane < TAIL, sq, jnp.zeros((L,), jnp.float32))

    accv[...] = acc
    pltpu.sync_copy(accv, out_hbm.at[wid])


def _sum_body(p_ref, o_ref):
    o_ref[0, 0] = jnp.sum(p_ref[...])


_sum_call = pl.pallas_call(
    _sum_body,
    out_shape=jax.ShapeDtypeStruct((1, 1), jnp.float32),
    out_specs=pl.BlockSpec(memory_space=pltpu.SMEM),
)


def kernel(input, edge_index, small_dists):
    ei_flat = edge_index.astype(jnp.int32).reshape(2 * N_EDGES)
    partials = _edge_partials(input, ei_flat, small_dists)
    return _sum_call(partials)[0, 0]


# speedup vs baseline: 1.0036x; 1.0036x over previous
import functools

import jax
import jax.numpy as jnp
from jax import lax
from jax.experimental import pallas as pl
from jax.experimental.pallas import tpu as pltpu
from jax.experimental.pallas import tpu_sc as plsc

N_NODES = 10000
N_EDGES = 160000
D_FEAT = 256

NC, NS, L = 2, 16, 16
NW = NC * NS
EPW = N_EDGES // NW
B = 32
NFULL = EPW // B
TAIL = EPW - NFULL * B
GROUPS = B // L
UNROLL = 8
DW = D_FEAT // 2
RPT = N_NODES // NS
RSTAGE = 25


def _rsqrt_newton(ss):
    ib = lax.bitcast_convert_type(ss, jnp.int32)
    seed = jnp.int32(0x5F3759DF) - lax.shift_right_logical(ib, 1)
    y = lax.bitcast_convert_type(seed, jnp.float32)
    for _ in range(3):
        y = y * (1.5 - 0.5 * ss * y * y)
    return y


def _group_sumsq(rows, row_idx, lane):
    def feat_body(k, carry):
        s0, s1 = carry
        col0 = lane + (k * UNROLL)
        for dd in range(UNROLL):
            col = jnp.bitwise_and(col0 + dd, DW - 1)
            w0 = plsc.load_gather(rows, [row_idx, col])
            w1 = plsc.load_gather(rows, [row_idx + 1, col])
            d = plsc.bitcast(w0, jnp.bfloat16) - plsc.bitcast(w1, jnp.bfloat16)
            d2 = plsc.bitcast(d * d, jnp.int32)
            lo = lax.bitcast_convert_type(jnp.left_shift(d2, 16), jnp.float32)
            hi = lax.bitcast_convert_type(
                jnp.bitwise_and(d2, jnp.int32(-65536)), jnp.float32)
            s0 = s0 + lo
            s1 = s1 + hi
        return s0, s1

    z = jnp.zeros((L,), jnp.float32)
    s0, s1 = lax.fori_loop(0, DW // UNROLL, feat_body, (z, z))
    return s0 + s1


def _edge_sqerr(ss, sd):
    ss = jnp.maximum(ss, 1e-30)
    dist = ss * _rsqrt_newton(ss)
    r = sd - dist
    return r * r


_mesh = plsc.VectorSubcoreMesh(core_axis_name="c", subcore_axis_name="s")


@functools.partial(
    pl.kernel,
    mesh=_mesh,
    compiler_params=pltpu.CompilerParams(use_tc_tiling_on_sc=False,
                                         needs_layout_passes=False),
    out_type=jax.ShapeDtypeStruct((NW, L), jnp.float32),
    scratch_types=[
        pltpu.VMEM_SHARED((N_NODES, DW), jnp.int32),
        pltpu.VMEM((2 * EPW + 32,), jnp.int32),
        pltpu.VMEM((EPW + 16,), jnp.float32),
        pltpu.VMEM((B, DW), jnp.int32),
        pltpu.VMEM((B, DW), jnp.int32),
        pltpu.VMEM((B, DW), jnp.int32),
        pltpu.VMEM((B, DW), jnp.int32),
        pltpu.VMEM((RSTAGE, D_FEAT), jnp.float32),
        pltpu.VMEM((RSTAGE, DW), jnp.int32),
        pltpu.VMEM((L,), jnp.float32),
        pltpu.SemaphoreType.DMA,
        pltpu.SemaphoreType.DMA,
        pltpu.SemaphoreType.DMA,
        pltpu.SemaphoreType.DMA,
    ],
)
def _edge_partials(x_hbm, idx_hbm, sd_hbm, out_hbm,
                   table, idx_all, sd_all,
                   rowsa0, rowsa1, rowsb0, rowsb1, stage, wstage, accv,
                   s0a, s1a, s0b, s1b):
    cid = lax.axis_index("c")
    sid = lax.axis_index("s")
    wid = sid * NC + cid
    e0 = pl.multiple_of(wid * EPW, 8)
    lane = lax.iota(jnp.int32, L)

    tbase = sid * RPT
    for cc in range(RPT // RSTAGE):
        nb = tbase + cc * RSTAGE
        pltpu.sync_copy(x_hbm.at[pl.ds(nb, RSTAGE)], stage)

        def row_body(r, carry):
            for k in range(D_FEAT // 32):
                a = stage[r, pl.ds(32 * k, 16)]
                b = stage[r, pl.ds(32 * k + 16, 16)]
                wv = plsc.bitcast(
                    plsc.pack(a, b, format=plsc.PackFormat.INTERLEAVED),
                    jnp.int32)
                wstage[r, pl.ds(16 * k, 16)] = wv
            return carry

        lax.fori_loop(0, RSTAGE, row_body, 0)
        pltpu.sync_copy(wstage, table.at[pl.ds(nb, RSTAGE)])
    plsc.subcore_barrier()

    rows = ((rowsa0, rowsa1), (rowsb0, rowsb1))
    sems = ((s0a, s1a), (s0b, s1b))

    pltpu.sync_copy(idx_hbm.at[pl.ds(2 * e0, 2 * EPW)],
                    idx_all.at[pl.ds(0, 2 * EPW)])
    pltpu.sync_copy(sd_hbm.at[pl.ds(e0, EPW)], sd_all.at[pl.ds(0, EPW)])

    def issue(t, parity, n):
        r0, r1 = rows[parity]
        sm0, sm1 = sems[parity]
        pltpu.async_copy(table.at[idx_all.at[pl.ds(t * 2 * B, n)]],
                         r0.at[pl.ds(0, n)], sm0)
        pltpu.async_copy(table.at[idx_all.at[pl.ds(t * 2 * B + n, n)]],
                         r1.at[pl.ds(0, n)], sm1)

    def wait(t, parity, n):
        r0, r1 = rows[parity]
        sm0, sm1 = sems[parity]
        pltpu.make_async_copy(table.at[idx_all.at[pl.ds(t * 2 * B, n)]],
                              r0.at[pl.ds(0, n)], sm0).wait()
        pltpu.make_async_copy(table.at[idx_all.at[pl.ds(t * 2 * B + n, n)]],
                              r1.at[pl.ds(0, n)], sm1).wait()

    def compute(t, parity, acc):
        for g in range(GROUPS):
            row_idx = lane * 2
            ss = _group_sumsq(rows[parity][g], row_idx, lane)
            sd = sd_all[pl.ds(t * B + g * L, L)]
            acc = acc + _edge_sqerr(ss, sd)
        return acc

    issue(0, 0, B)

    def outer_body(i, acc):
        tb = i * 2
        for par in range(2):
            t = tb + par

            @pl.when(t + 1 < NFULL)
            def _():
                issue(t + 1, 1 - par, B)

            wait(t, par, B)
            acc = compute(t, par, acc)
        return acc

    acc = lax.fori_loop(0, NFULL // 2, outer_body,
                        jnp.zeros((L,), jnp.float32))

    if NFULL % 2:
        t_last = NFULL - 1
        wait(t_last, t_last % 2, B)
        acc = compute(t_last, t_last % 2, acc)

    tpar = NFULL % 2
    tr0 = rows[tpar][0]
    tsm = sems[tpar][0]
    pltpu.async_copy(table.at[idx_all.at[pl.ds(NFULL * 2 * B, 2 * TAIL)]],
                     tr0.at[pl.ds(0, 2 * TAIL)], tsm)
    pltpu.make_async_copy(table.at[idx_all.at[pl.ds(NFULL * 2 * B, 2 * TAIL)]],
                          tr0.at[pl.ds(0, 2 * TAIL)], tsm).wait()
    row_idx = jnp.bitwise_and(lane, TAIL - 1) * 2
    ss = _group_sumsq(tr0, row_idx, lane)
    sd = sd_all[pl.ds(NFULL * B, L)]
    sq = _edge_sqerr(ss, sd)
    acc = acc + jnp.where(lane < TAIL, sq, jnp.zeros((L,), jnp.float32))

    accv[...] = acc
    pltpu.sync_copy(accv, out_hbm.at[wid])


def _sum_body(p_ref, o_ref):
    o_ref[0, 0] = jnp.sum(p_ref[...])


_sum_call = pl.pallas_call(
    _sum_body,
    out_shape=jax.ShapeDtypeStruct((1, 1), jnp.float32),
    out_specs=pl.BlockSpec(memory_space=pltpu.SMEM),
)


def kernel(input, edge_index, small_dists):
    ei_flat = edge_index.astype(jnp.int32).reshape(2 * N_EDGES)
    partials = _edge_partials(input, ei_flat, small_dists)
    return _sum_call(partials)[0, 0]


# speedup vs baseline: 1.5281x; 1.5226x over previous
import functools

import jax
import jax.numpy as jnp
from jax import lax
from jax.experimental import pallas as pl
from jax.experimental.pallas import tpu as pltpu
from jax.experimental.pallas import tpu_sc as plsc

N_NODES = 10000
N_EDGES = 160000
D_FEAT = 256

NC, NS, L = 2, 16, 16
NW = NC * NS
EPW = N_EDGES // NW
B = 32
NFULL = EPW // B
TAIL = EPW - NFULL * B
GROUPS = B // L
UNROLL = 8
DW = D_FEAT // 2
RPT = N_NODES // NS
RSTAGE = 25


def _rsqrt_newton(ss):
    ib = lax.bitcast_convert_type(ss, jnp.int32)
    seed = jnp.int32(0x5F3759DF) - lax.shift_right_logical(ib, 1)
    y = lax.bitcast_convert_type(seed, jnp.float32)
    for _ in range(3):
        y = y * (1.5 - 0.5 * ss * y * y)
    return y


def _group_sumsq(rows0, rows1, row_idx, lane):
    def feat_body(k, carry):
        s0, s1 = carry
        col0 = lane + (k * UNROLL)
        for dd in range(UNROLL):
            col = jnp.bitwise_and(col0 + dd, DW - 1)
            w0 = plsc.load_gather(rows0, [row_idx, col])
            w1 = plsc.load_gather(rows1, [row_idx, col])
            d = plsc.bitcast(w0, jnp.bfloat16) - plsc.bitcast(w1, jnp.bfloat16)
            d2 = plsc.bitcast(d * d, jnp.int32)
            lo = lax.bitcast_convert_type(jnp.left_shift(d2, 16), jnp.float32)
            hi = lax.bitcast_convert_type(
                jnp.bitwise_and(d2, jnp.int32(-65536)), jnp.float32)
            s0 = s0 + lo
            s1 = s1 + hi
        return s0, s1

    z = jnp.zeros((L,), jnp.float32)
    s0, s1 = lax.fori_loop(0, DW // UNROLL, feat_body, (z, z))
    return s0 + s1


def _edge_sqerr(ss, sd):
    ss = jnp.maximum(ss, 1e-30)
    dist = ss * _rsqrt_newton(ss)
    r = sd - dist
    return r * r


_mesh = plsc.VectorSubcoreMesh(core_axis_name="c", subcore_axis_name="s")


@functools.partial(
    pl.kernel,
    mesh=_mesh,
    compiler_params=pltpu.CompilerParams(use_tc_tiling_on_sc=False,
                                         needs_layout_passes=False),
    out_type=jax.ShapeDtypeStruct((NW, L), jnp.float32),
    scratch_types=[
        pltpu.VMEM_SHARED((N_NODES, DW), jnp.int32),
        pltpu.VMEM((EPW + 16,), jnp.int32),
        pltpu.VMEM((EPW + 16,), jnp.int32),
        pltpu.VMEM((EPW + 16,), jnp.float32),
        pltpu.VMEM((B, DW), jnp.int32),
        pltpu.VMEM((B, DW), jnp.int32),
        pltpu.VMEM((B, DW), jnp.int32),
        pltpu.VMEM((B, DW), jnp.int32),
        pltpu.VMEM((RSTAGE, D_FEAT), jnp.float32),
        pltpu.VMEM((RSTAGE, DW), jnp.int32),
        pltpu.VMEM((L,), jnp.float32),
        pltpu.SemaphoreType.DMA,
        pltpu.SemaphoreType.DMA,
        pltpu.SemaphoreType.DMA,
        pltpu.SemaphoreType.DMA,
    ],
)
def _edge_partials(x_hbm, idx0_hbm, idx1_hbm, sd_hbm, out_hbm,
                   table, idx0_all, idx1_all, sd_all,
                   rows0a, rows1a, rows0b, rows1b, stage, wstage, accv,
                   s0a, s1a, s0b, s1b):
    cid = lax.axis_index("c")
    sid = lax.axis_index("s")
    wid = sid * NC + cid
    e0 = pl.multiple_of(wid * EPW, 8)
    lane = lax.iota(jnp.int32, L)

    tbase = sid * RPT
    for cc in range(RPT // RSTAGE):
        nb = tbase + cc * RSTAGE
        pltpu.sync_copy(x_hbm.at[pl.ds(nb, RSTAGE)], stage)

        def row_body(r, carry):
            for k in range(D_FEAT // 32):
                a = stage[r, pl.ds(32 * k, 16)]
                b = stage[r, pl.ds(32 * k + 16, 16)]
                wv = plsc.bitcast(
                    plsc.pack(a, b, format=plsc.PackFormat.INTERLEAVED),
                    jnp.int32)
                wstage[r, pl.ds(16 * k, 16)] = wv
            return carry

        lax.fori_loop(0, RSTAGE, row_body, 0)
        pltpu.sync_copy(wstage, table.at[pl.ds(nb, RSTAGE)])
    plsc.subcore_barrier()

    rows = ((rows0a, rows1a), (rows0b, rows1b))
    sems = ((s0a, s1a), (s0b, s1b))

    pltpu.sync_copy(idx0_hbm.at[pl.ds(e0, EPW)], idx0_all.at[pl.ds(0, EPW)])
    pltpu.sync_copy(idx1_hbm.at[pl.ds(e0, EPW)], idx1_all.at[pl.ds(0, EPW)])
    pltpu.sync_copy(sd_hbm.at[pl.ds(e0, EPW)], sd_all.at[pl.ds(0, EPW)])

    def issue(t, parity, n):
        r0, r1 = rows[parity]
        sm0, sm1 = sems[parity]
        pltpu.async_copy(table.at[idx0_all.at[pl.ds(t * B, n)]],
                         r0.at[pl.ds(0, n)], sm0)
        pltpu.async_copy(table.at[idx1_all.at[pl.ds(t * B, n)]],
                         r1.at[pl.ds(0, n)], sm1)

    def wait(t, parity, n):
        r0, r1 = rows[parity]
        sm0, sm1 = sems[parity]
        pltpu.make_async_copy(table.at[idx0_all.at[pl.ds(t * B, n)]],
                              r0.at[pl.ds(0, n)], sm0).wait()
        pltpu.make_async_copy(table.at[idx1_all.at[pl.ds(t * B, n)]],
                              r1.at[pl.ds(0, n)], sm1).wait()

    def compute(t, parity, acc):
        r0, r1 = rows[parity]
        for g in range(GROUPS):
            row_idx = lane + (g * L)
            ss = _group_sumsq(r0, r1, row_idx, lane)
            sd = sd_all[pl.ds(t * B + g * L, L)]
            acc = acc + _edge_sqerr(ss, sd)
        return acc

    issue(0, 0, B)

    def outer_body(i, acc):
        tb = i * 2
        for par in range(2):
            t = tb + par

            @pl.when(t + 1 < NFULL)
            def _():
                issue(t + 1, 1 - par, B)

            wait(t, par, B)
            acc = compute(t, par, acc)
        return acc

    acc = lax.fori_loop(0, NFULL // 2, outer_body,
                        jnp.zeros((L,), jnp.float32))

    if NFULL % 2:
        t_last = NFULL - 1
        wait(t_last, t_last % 2, B)
        acc = compute(t_last, t_last % 2, acc)

    tpar = NFULL % 2
    issue(NFULL, tpar, TAIL)
    wait(NFULL, tpar, TAIL)
    r0, r1 = rows[tpar]
    row_idx = jnp.bitwise_and(lane, TAIL - 1)
    ss = _group_sumsq(r0, r1, row_idx, lane)
    sd = sd_all[pl.ds(NFULL * B, L)]
    sq = _edge_sqerr(ss, sd)
    acc = acc + jnp.where(lane < TAIL, sq, jnp.zeros((L,), jnp.float32))

    accv[...] = acc
    pltpu.sync_copy(accv, out_hbm.at[wid])


def _sum_body(p_ref, o_ref):
    o_ref[0, 0] = jnp.sum(p_ref[...])


_sum_call = pl.pallas_call(
    _sum_body,
    out_shape=jax.ShapeDtypeStruct((1, 1), jnp.float32),
    out_specs=pl.BlockSpec(memory_space=pltpu.SMEM),
)


def kernel(input, edge_index, small_dists):
    ei = edge_index.astype(jnp.int32)
    idx0 = ei[:, 0]
    idx1 = ei[:, 1]
    partials = _edge_partials(input, idx0, idx1, small_dists)
    return _sum_call(partials)[0, 0]
